# baseline (speedup 1.0000x reference)
import jax
import jax.numpy as jnp
from jax.experimental import pallas as pl
from jax.experimental.pallas import tpu as pltpu

IN_DIM = 1024
OUT_DIM = 1000
NUM_EXPERT = 16
TOP_K = 12
HIDDEN = 256
N_TOK = 2048
TILE_N = 512
EH = NUM_EXPERT * HIDDEN
EE = NUM_EXPERT * NUM_EXPERT


def _moe_kernel(mod_ref, x_ref, wg_ref, W1_ref, W2c_ref, Wout_ref, bout_ref,
                y_ref, gates_ref, load_ref, H_ref):
    i = pl.program_id(0)
    e = pl.program_id(1)
    f32 = jnp.float32
    x = x_ref[...]

    @pl.when(e == 0)
    def _gating():
        wg = jnp.where(mod_ref[0] == 0,
                       wg_ref[:IN_DIM], wg_ref[IN_DIM:])
        logits = jnp.dot(x, wg, preferred_element_type=f32)

        row16 = jax.lax.broadcasted_iota(jnp.int32, (NUM_EXPERT, EE), 0)
        col = jax.lax.broadcasted_iota(jnp.int32, (NUM_EXPERT, EE), 1)
        q_of_c = col >> 4
        j_of_c = col & 15
        R16 = (row16 == q_of_c).astype(f32)
        T16 = (row16 == j_of_c).astype(f32)
        rep_q = jnp.dot(logits, R16, preferred_element_type=f32,
                        precision=jax.lax.Precision.HIGHEST)
        rep_j = jnp.dot(logits, T16, preferred_element_type=f32,
                        precision=jax.lax.Precision.HIGHEST)
        colv = jax.lax.broadcasted_iota(jnp.int32, (TILE_N, EE), 1)
        tie = ((colv & 15) < (colv >> 4))
        cmp = (rep_j > rep_q).astype(f32) + jnp.where(
            (rep_j == rep_q) & tie, 1.0, 0.0)
        S = (q_of_c.T == jax.lax.broadcasted_iota(
            jnp.int32, (EE, NUM_EXPERT), 1)).astype(f32)
        rank = jnp.dot(cmp, S, preferred_element_type=f32)
        sel = rank < TOP_K

        m = jnp.max(logits, axis=1, keepdims=True)
        ex = jnp.where(sel, jnp.exp(logits - m), 0.0)
        g = ex / jnp.sum(ex, axis=1, keepdims=True)
        gates_ref[...] = g

        @pl.when(i == 0)
        def _():
            load_ref[...] = jnp.zeros_like(load_ref)
        load_ref[...] += jnp.sum((g > 0).astype(f32), axis=0, keepdims=True)

    H_col = jnp.maximum(
        jnp.dot(x, W1_ref[0], preferred_element_type=f32), 0.0)
    for ee in range(NUM_EXPERT):
        @pl.when(e == ee)
        def _(ee=ee):
            H_ref[:, ee * HIDDEN:(ee + 1) * HIDDEN] = H_col.astype(
                jnp.bfloat16)

    @pl.when(e == NUM_EXPERT - 1)
    def _tail():
        g = gates_ref[...]
        G = jnp.broadcast_to(g[:, :, None],
                             (TILE_N, NUM_EXPERT, HIDDEN)).reshape(TILE_N, EH)
        out = jnp.dot(H_ref[...] * G, W2c_ref[...],
                      preferred_element_type=f32)
        yin = jnp.maximum(out, 0.0) + x
        y_ref[...] = (jnp.dot(yin, Wout_ref[...],
                              preferred_element_type=f32) + bout_ref[...])


def kernel(x, modality, w_gates, W1, b1, W2, b2, Wout, bout):
    mod = jnp.asarray(modality, jnp.int32).reshape(1)
    W2c = W2.reshape(EH, IN_DIM)
    n_tiles = N_TOK // TILE_N
    y, gates, load = pl.pallas_call(
        _moe_kernel,
        grid=(n_tiles, NUM_EXPERT),
        in_specs=[
            pl.BlockSpec(memory_space=pltpu.SMEM),
            pl.BlockSpec((TILE_N, IN_DIM), lambda i, e: (i, 0)),
            pl.BlockSpec((2 * IN_DIM, NUM_EXPERT), lambda i, e: (0, 0)),
            pl.BlockSpec((1, IN_DIM, HIDDEN), lambda i, e: (e, 0, 0)),
            pl.BlockSpec((EH, IN_DIM), lambda i, e: (0, 0)),
            pl.BlockSpec((IN_DIM, OUT_DIM), lambda i, e: (0, 0)),
            pl.BlockSpec((1, OUT_DIM), lambda i, e: (0, 0)),
        ],
        out_specs=[
            pl.BlockSpec((TILE_N, OUT_DIM), lambda i, e: (i, 0)),
            pl.BlockSpec((TILE_N, NUM_EXPERT), lambda i, e: (i, 0)),
            pl.BlockSpec((1, NUM_EXPERT), lambda i, e: (0, 0)),
        ],
        out_shape=[
            jax.ShapeDtypeStruct((N_TOK, OUT_DIM), jnp.float32),
            jax.ShapeDtypeStruct((N_TOK, NUM_EXPERT), jnp.float32),
            jax.ShapeDtypeStruct((1, NUM_EXPERT), jnp.float32),
        ],
        scratch_shapes=[pltpu.VMEM((TILE_N, EH), jnp.bfloat16)],
    )(mod, x, w_gates.reshape(2 * IN_DIM, NUM_EXPERT), W1, W2c, Wout,
      bout.reshape(1, OUT_DIM))
    return (y, gates, load.reshape(NUM_EXPERT))


# speedup vs baseline: 1.4694x; 1.4694x over previous
import jax
import jax.numpy as jnp
from jax.experimental import pallas as pl
from jax.experimental.pallas import tpu as pltpu

IN_DIM = 1024
OUT_DIM = 1000
NUM_EXPERT = 16
TOP_K = 12
HIDDEN = 256
N_TOK = 2048
TILE_N = 256
EH = NUM_EXPERT * HIDDEN
EE = NUM_EXPERT * NUM_EXPERT


def _moe_kernel(mod_ref, x_ref, wg_ref, W1_ref, W2c_ref, Wout_ref, bout_ref,
                y_ref, gates_ref, load_ref, H_ref):
    i = pl.program_id(0)
    x = x_ref[...]
    f32 = jnp.float32

    wg = jnp.where(mod_ref[0] == 0, wg_ref[:IN_DIM], wg_ref[IN_DIM:])
    logits = jnp.dot(x, wg, preferred_element_type=f32)

    row16 = jax.lax.broadcasted_iota(jnp.int32, (NUM_EXPERT, EE), 0)
    col = jax.lax.broadcasted_iota(jnp.int32, (NUM_EXPERT, EE), 1)
    e_of_c = col >> 4
    j_of_c = col & 15
    R16 = (row16 == e_of_c).astype(f32)
    T16 = (row16 == j_of_c).astype(f32)
    rep_e = jnp.dot(logits, R16, preferred_element_type=f32,
                    precision=jax.lax.Precision.HIGHEST)
    rep_j = jnp.dot(logits, T16, preferred_element_type=f32,
                    precision=jax.lax.Precision.HIGHEST)
    colv = jax.lax.broadcasted_iota(jnp.int32, (TILE_N, EE), 1)
    tie = ((colv & 15) < (colv >> 4))
    cmp = (rep_j > rep_e).astype(f32) + jnp.where(
        (rep_j == rep_e) & tie, 1.0, 0.0)
    S = (e_of_c.T == jax.lax.broadcasted_iota(
        jnp.int32, (EE, NUM_EXPERT), 1)).astype(f32)
    rank = jnp.dot(cmp, S, preferred_element_type=f32)
    sel = rank < TOP_K

    m = jnp.max(logits, axis=1, keepdims=True)
    ex = jnp.where(sel, jnp.exp(logits - m), 0.0)
    g = ex / jnp.sum(ex, axis=1, keepdims=True)
    gates_ref[...] = g

    @pl.when(i == 0)
    def _():
        load_ref[...] = jnp.zeros_like(load_ref)
    load_ref[...] += jnp.sum((g > 0).astype(f32), axis=0, keepdims=True)

    for e in range(NUM_EXPERT):
        H_ref[:, e * HIDDEN:(e + 1) * HIDDEN] = jnp.maximum(
            jnp.dot(x, W1_ref[e], preferred_element_type=f32),
            0.0).astype(jnp.bfloat16)
    G = jnp.broadcast_to(g[:, :, None],
                         (TILE_N, NUM_EXPERT, HIDDEN)).reshape(TILE_N, EH)
    out = jnp.dot(H_ref[...] * G, W2c_ref[...],
                  preferred_element_type=f32)

    yin = jnp.maximum(out, 0.0) + x
    y_ref[...] = (jnp.dot(yin, Wout_ref[...], preferred_element_type=f32)
                  + bout_ref[...])


def kernel(x, modality, w_gates, W1, b1, W2, b2, Wout, bout):
    mod = jnp.asarray(modality, jnp.int32).reshape(1)
    W2c = W2.reshape(EH, IN_DIM)
    n_tiles = N_TOK // TILE_N
    y, gates, load = pl.pallas_call(
        _moe_kernel,
        grid=(n_tiles,),
        in_specs=[
            pl.BlockSpec(memory_space=pltpu.SMEM),
            pl.BlockSpec((TILE_N, IN_DIM), lambda i: (i, 0)),
            pl.BlockSpec((2 * IN_DIM, NUM_EXPERT), lambda i: (0, 0)),
            pl.BlockSpec((NUM_EXPERT, IN_DIM, HIDDEN), lambda i: (0, 0, 0)),
            pl.BlockSpec((EH, IN_DIM), lambda i: (0, 0)),
            pl.BlockSpec((IN_DIM, OUT_DIM), lambda i: (0, 0)),
            pl.BlockSpec((1, OUT_DIM), lambda i: (0, 0)),
        ],
        out_specs=[
            pl.BlockSpec((TILE_N, OUT_DIM), lambda i: (i, 0)),
            pl.BlockSpec((TILE_N, NUM_EXPERT), lambda i: (i, 0)),
            pl.BlockSpec((1, NUM_EXPERT), lambda i: (0, 0)),
        ],
        out_shape=[
            jax.ShapeDtypeStruct((N_TOK, OUT_DIM), jnp.float32),
            jax.ShapeDtypeStruct((N_TOK, NUM_EXPERT), jnp.float32),
            jax.ShapeDtypeStruct((1, NUM_EXPERT), jnp.float32),
        ],
        scratch_shapes=[pltpu.VMEM((TILE_N, EH), jnp.bfloat16)],
    )(mod, x, w_gates.reshape(2 * IN_DIM, NUM_EXPERT), W1, W2c, Wout,
      bout.reshape(1, OUT_DIM))
    return (y, gates, load.reshape(NUM_EXPERT))


# speedup vs baseline: 1.5700x; 1.0685x over previous
import jax
import jax.numpy as jnp
from jax.experimental import pallas as pl
from jax.experimental.pallas import tpu as pltpu

IN_DIM = 1024
OUT_DIM = 1000
NUM_EXPERT = 16
TOP_K = 12
HIDDEN = 256
N_TOK = 2048
TILE_N = 512
EH = NUM_EXPERT * HIDDEN
EE = NUM_EXPERT * NUM_EXPERT


def _moe_kernel(x_ref, wg_ref, W1_ref, W2c_ref, Wout_ref, bout_ref,
                y_ref, gates_ref, load_ref, H_ref):
    i = pl.program_id(0)
    x = x_ref[...]
    f32 = jnp.float32

    logits = jnp.dot(x, wg_ref[...], preferred_element_type=f32)

    row16 = jax.lax.broadcasted_iota(jnp.int32, (NUM_EXPERT, EE), 0)
    col = jax.lax.broadcasted_iota(jnp.int32, (NUM_EXPERT, EE), 1)
    e_of_c = col >> 4
    j_of_c = col & 15
    R16 = (row16 == e_of_c).astype(f32)
    T16 = (row16 == j_of_c).astype(f32)
    rep_e = jnp.dot(logits, R16, preferred_element_type=f32,
                    precision=jax.lax.Precision.HIGHEST)
    rep_j = jnp.dot(logits, T16, preferred_element_type=f32,
                    precision=jax.lax.Precision.HIGHEST)
    colv = jax.lax.broadcasted_iota(jnp.int32, (TILE_N, EE), 1)
    tie = ((colv & 15) < (colv >> 4))
    cmp = (rep_j > rep_e).astype(f32) + jnp.where(
        (rep_j == rep_e) & tie, 1.0, 0.0)
    S = (e_of_c.T == jax.lax.broadcasted_iota(
        jnp.int32, (EE, NUM_EXPERT), 1)).astype(f32)
    rank = jnp.dot(cmp, S, preferred_element_type=f32)
    sel = rank < TOP_K

    m = jnp.max(logits, axis=1, keepdims=True)
    ex = jnp.where(sel, jnp.exp(logits - m), 0.0)
    g = ex / jnp.sum(ex, axis=1, keepdims=True)
    gates_ref[...] = g

    @pl.when(i == 0)
    def _():
        load_ref[...] = jnp.zeros_like(load_ref)
    load_ref[...] += jnp.sum((g > 0).astype(f32), axis=0, keepdims=True)

    for e in range(NUM_EXPERT):
        H_ref[:, e * HIDDEN:(e + 1) * HIDDEN] = jnp.maximum(
            jnp.dot(x, W1_ref[e], preferred_element_type=f32), 0.0)
    G = jnp.broadcast_to(g[:, :, None],
                         (TILE_N, NUM_EXPERT, HIDDEN)).reshape(TILE_N, EH)
    out = jnp.dot(H_ref[...] * G, W2c_ref[...],
                  preferred_element_type=f32)

    yin = jnp.maximum(out, 0.0) + x
    y_ref[...] = (jnp.dot(yin, Wout_ref[...], preferred_element_type=f32)
                  + bout_ref[...])


def kernel(x, modality, w_gates, W1, b1, W2, b2, Wout, bout):
    wg = w_gates[modality]
    W2c = W2.reshape(EH, IN_DIM)
    n_tiles = N_TOK // TILE_N
    y, gates, load = pl.pallas_call(
        _moe_kernel,
        grid=(n_tiles,),
        in_specs=[
            pl.BlockSpec((TILE_N, IN_DIM), lambda i: (i, 0)),
            pl.BlockSpec((IN_DIM, NUM_EXPERT), lambda i: (0, 0)),
            pl.BlockSpec((NUM_EXPERT, IN_DIM, HIDDEN), lambda i: (0, 0, 0)),
            pl.BlockSpec((EH, IN_DIM), lambda i: (0, 0)),
            pl.BlockSpec((IN_DIM, OUT_DIM), lambda i: (0, 0)),
            pl.BlockSpec((1, OUT_DIM), lambda i: (0, 0)),
        ],
        out_specs=[
            pl.BlockSpec((TILE_N, OUT_DIM), lambda i: (i, 0)),
            pl.BlockSpec((TILE_N, NUM_EXPERT), lambda i: (i, 0)),
            pl.BlockSpec((1, NUM_EXPERT), lambda i: (0, 0)),
        ],
        out_shape=[
            jax.ShapeDtypeStruct((N_TOK, OUT_DIM), jnp.float32),
            jax.ShapeDtypeStruct((N_TOK, NUM_EXPERT), jnp.float32),
            jax.ShapeDtypeStruct((1, NUM_EXPERT), jnp.float32),
        ],
        scratch_shapes=[pltpu.VMEM((TILE_N, EH), jnp.float32)],
    )(x, wg, W1, W2c, Wout, bout.reshape(1, OUT_DIM))
    return (y, gates, load.reshape(NUM_EXPERT))
